# baseline (speedup 1.0000x reference)
import functools

import jax
import jax.numpy as jnp
from jax import lax
from jax.experimental import pallas as pl
from jax.experimental.pallas import tpu as pltpu
from jax.experimental.pallas import tpu_sc as plsc

PRE_SEQ_LEN = 128
HIDDEN = 1024
OUT_DIM = 24 * HIDDEN
BATCH = 16
N_TOK = BATCH * PRE_SEQ_LEN
_NQ = 4
_QW = OUT_DIM // _NQ


_DT = 3072
_NT = OUT_DIM // _DT
_TPQ = _QW // _DT


def _mlp_body(e_ref, w1_ref, b1_ref, w2_ref, b2_ref, p_ref, h_ref):
    @pl.when(pl.program_id(0) == 0)
    def _():
        h = jnp.dot(e_ref[...], w1_ref[...], preferred_element_type=jnp.float32)
        h_ref[...] = jnp.tanh(h + b1_ref[...])

    p = jnp.dot(h_ref[...], w2_ref[...], preferred_element_type=jnp.float32)
    p_ref[...] = (p + b2_ref[...])[None]


def _mlp(emb_table, W1, b1, W2, b2):
    return pl.pallas_call(
        _mlp_body,
        grid=(_NT,),
        in_specs=[
            pl.BlockSpec((PRE_SEQ_LEN, HIDDEN), lambda j: (0, 0)),
            pl.BlockSpec((HIDDEN, HIDDEN), lambda j: (0, 0)),
            pl.BlockSpec((1, HIDDEN), lambda j: (0, 0)),
            pl.BlockSpec((HIDDEN, _DT), lambda j: (0, j)),
            pl.BlockSpec((1, _DT), lambda j: (0, j)),
        ],
        out_specs=pl.BlockSpec(
            (1, PRE_SEQ_LEN, _DT), lambda j: (j // _TPQ, 0, j % _TPQ)
        ),
        out_shape=jax.ShapeDtypeStruct((_NQ, PRE_SEQ_LEN, _QW), jnp.float32),
        scratch_shapes=[pltpu.VMEM((PRE_SEQ_LEN, HIDDEN), jnp.float32)],
    )(emb_table, W1, b1, W2, b2)


_NC = 2
_NS = 16
_TPW = N_TOK // _NS
_INFLIGHT = 8


def _gather_body(p_hbm, idx_hbm, out_hbm, idx_v, sp, wsem):
    cid = lax.axis_index("c")
    sid = lax.axis_index("s")
    tok0 = sid * _TPW

    @pl.when(sid == 0)
    def _():
        pltpu.sync_copy(p_hbm.at[cid * 2], sp.at[0])

    @pl.when(sid == 1)
    def _():
        pltpu.sync_copy(p_hbm.at[cid * 2 + 1], sp.at[1])

    pltpu.sync_copy(idx_hbm.at[pl.ds(tok0, _TPW)], idx_v)
    plsc.subcore_barrier()

    def group(g, carry):
        v = idx_v[pl.ds(g * 16, 16)]
        for k in range(16):
            t = v[k]
            for h in range(2):
                pltpu.async_copy(
                    sp.at[h].at[pl.ds(t, 1)],
                    out_hbm.at[pl.ds(tok0 + g * 16 + k, 1), cid * 2 + h],
                    wsem,
                )
        return carry

    lax.fori_loop(0, _TPW // 16, group, 0)

    def drain(j, carry):
        pltpu.make_async_copy(
            sp.at[0].at[pl.ds(0, 1)],
            out_hbm.at[pl.ds(tok0, 1), 0],
            wsem,
        ).wait()
        return carry

    lax.fori_loop(0, 2 * _TPW, drain, 0)


@functools.cache
def _make_gather():
    return pl.kernel(
        _gather_body,
        out_type=jax.ShapeDtypeStruct((N_TOK, _NQ, _QW), jnp.float32),
        mesh=plsc.VectorSubcoreMesh(
            core_axis_name="c", subcore_axis_name="s",
            num_cores=_NC, num_subcores=_NS,
        ),
        scratch_types=[
            pltpu.VMEM((_TPW,), jnp.int32),
            pltpu.VMEM_SHARED((2, PRE_SEQ_LEN, _QW), jnp.float32),
            pltpu.SemaphoreType.DMA,
        ],
    )


def kernel(prefix, emb_table, W1, b1, W2, b2):
    P = _mlp(emb_table, W1, b1.reshape(1, HIDDEN), W2, b2.reshape(1, OUT_DIM))
    idx = prefix.reshape(N_TOK).astype(jnp.int32)
    out = _make_gather()(P, idx)
    return out.reshape(BATCH, PRE_SEQ_LEN, OUT_DIM)


# speedup vs baseline: 1.8682x; 1.8682x over previous
import functools

import jax
import jax.numpy as jnp
from jax import lax
from jax.experimental import pallas as pl
from jax.experimental.pallas import tpu as pltpu
from jax.experimental.pallas import tpu_sc as plsc

PRE_SEQ_LEN = 128
HIDDEN = 1024
OUT_DIM = 24 * HIDDEN
BATCH = 16
N_TOK = BATCH * PRE_SEQ_LEN


_DT = 3072
_NT = OUT_DIM // _DT


def _mlp_body(e_ref, w1_ref, b1_ref, w2_ref, b2_ref, p_ref, h_ref):
    @pl.when(pl.program_id(0) == 0)
    def _():
        h = jnp.dot(e_ref[...], w1_ref[...], preferred_element_type=jnp.float32)
        h_ref[...] = jnp.tanh(h + b1_ref[...])

    p = jnp.dot(h_ref[...], w2_ref[...], preferred_element_type=jnp.float32)
    p_ref[...] = p + b2_ref[...]


def _mlp(emb_table, W1, b1, W2, b2):
    return pl.pallas_call(
        _mlp_body,
        grid=(_NT,),
        in_specs=[
            pl.BlockSpec((PRE_SEQ_LEN, HIDDEN), lambda j: (0, 0)),
            pl.BlockSpec((HIDDEN, HIDDEN), lambda j: (0, 0)),
            pl.BlockSpec((1, HIDDEN), lambda j: (0, 0)),
            pl.BlockSpec((HIDDEN, _DT), lambda j: (0, j)),
            pl.BlockSpec((1, _DT), lambda j: (0, j)),
        ],
        out_specs=pl.BlockSpec((PRE_SEQ_LEN, _DT), lambda j: (0, j)),
        out_shape=jax.ShapeDtypeStruct((PRE_SEQ_LEN, OUT_DIM), jnp.float32),
        scratch_shapes=[pltpu.VMEM((PRE_SEQ_LEN, HIDDEN), jnp.float32)],
    )(emb_table, W1, b1, W2, b2)


_NC = 2
_NS = 16
_NW = _NC * _NS
_TPW = N_TOK // _NW
_CH = 2
_NCHUNK = _TPW // _CH
_NBUF = 2


def _gather_body(p_hbm, idx_hbm, out_hbm, idx_v, rows_v, *sems):
    gsem = sems[:_NBUF]
    wsem = sems[_NBUF:]
    wid = lax.axis_index("s") * _NC + lax.axis_index("c")
    tok0 = wid * _TPW
    pltpu.sync_copy(idx_hbm.at[pl.ds(wid * _NCHUNK, _NCHUNK)], idx_v)

    def g_start(c, b):
        pltpu.async_copy(p_hbm.at[idx_v.at[c]], rows_v.at[b], gsem[b])

    def g_wait(b):
        pltpu.make_async_copy(p_hbm.at[idx_v.at[0]], rows_v.at[b], gsem[b]).wait()

    def w_start(c, b):
        pltpu.async_copy(
            rows_v.at[b], out_hbm.at[pl.ds(tok0 + c * _CH, _CH)], wsem[b]
        )

    def w_wait(c, b):
        pltpu.make_async_copy(
            rows_v.at[b], out_hbm.at[pl.ds(tok0 + c * _CH, _CH)], wsem[b]
        ).wait()

    for b in range(_NBUF):
        g_start(b, b)

    def outer(i, carry):
        c = i * _NBUF
        for b in range(_NBUF):
            g_wait(b)
            w_start(c + b, b)
            w_wait(c + b, b)
            g_start(c + b + _NBUF, b)
        return carry

    lax.fori_loop(0, _NCHUNK // _NBUF - 1, outer, 0)

    for b in range(_NBUF):
        c = _NCHUNK - _NBUF + b
        g_wait(b)
        w_start(c, b)
    for b in range(_NBUF):
        w_wait(_NCHUNK - _NBUF + b, b)


@functools.cache
def _make_gather():
    return pl.kernel(
        _gather_body,
        out_type=jax.ShapeDtypeStruct((N_TOK, OUT_DIM), jnp.float32),
        mesh=plsc.VectorSubcoreMesh(
            core_axis_name="c", subcore_axis_name="s",
            num_cores=_NC, num_subcores=_NS,
        ),
        scratch_types=[
            pltpu.VMEM((_NCHUNK, _CH), jnp.int32),
            pltpu.VMEM((_NBUF, _CH, OUT_DIM), jnp.float32),
        ] + [pltpu.SemaphoreType.DMA] * (2 * _NBUF),
    )


def kernel(prefix, emb_table, W1, b1, W2, b2):
    P = _mlp(emb_table, W1, b1.reshape(1, HIDDEN), W2, b2.reshape(1, OUT_DIM))
    idx = prefix.reshape(N_TOK).astype(jnp.int32)
    out = _make_gather()(P, idx.reshape(N_TOK // _CH, _CH))
    return out.reshape(BATCH, PRE_SEQ_LEN, OUT_DIM)
